# initial kernel scaffold (unmeasured)
import jax
import jax.numpy as jnp
from jax import lax
from jax.experimental import pallas as pl
from jax.experimental.pallas import tpu as pltpu


def kernel(
    x,
):
    def body(*refs):
        pass

    out_shape = jax.ShapeDtypeStruct(..., jnp.float32)
    return pl.pallas_call(body, out_shape=out_shape)(...)


# baseline (device time: 328757 ns/iter reference)
import jax
import jax.numpy as jnp
from jax import lax
from jax.experimental import pallas as pl
from jax.experimental.pallas import tpu as pltpu

N_RING = 4
M = 4096
N_COLS = 4096
CHUNK = N_COLS // N_RING


def kernel(x):
    x_bf = x[0].astype(jnp.bfloat16)

    def body(x_hbm, out_ref, comm, stage, copy_sem, send_sems, recv_sems):
        my_x = lax.axis_index("x")
        my_y = lax.axis_index("y")
        my_z = lax.axis_index("z")
        right = lax.rem(my_y + 1, N_RING)
        left = lax.rem(my_y + N_RING - 1, N_RING)

        barrier = pltpu.get_barrier_semaphore()
        for nbr in (left, right):
            pl.semaphore_signal(
                barrier, inc=1,
                device_id=(my_x, nbr, my_z),
                device_id_type=pl.DeviceIdType.MESH,
            )
        pl.semaphore_wait(barrier, 2)

        c0 = lax.rem(my_y + N_RING - 1, N_RING)
        cp = pltpu.make_async_copy(
            x_hbm.at[:, pl.ds(c0 * CHUNK, CHUNK)], comm.at[0], copy_sem
        )
        cp.start()
        cp.wait()

        for s in range(N_RING - 1):
            rdma = pltpu.make_async_remote_copy(
                src_ref=comm.at[s],
                dst_ref=comm.at[s + 1],
                send_sem=send_sems.at[s],
                recv_sem=recv_sems.at[s],
                device_id=(my_x, right, my_z),
                device_id_type=pl.DeviceIdType.MESH,
            )
            rdma.start()
            c = lax.rem(my_y + 2 * N_RING - 2 - s, N_RING)
            cpl = pltpu.make_async_copy(
                x_hbm.at[:, pl.ds(c * CHUNK, CHUNK)], stage, copy_sem
            )
            cpl.start()
            cpl.wait()
            rdma.wait()
            if s < N_RING - 2:
                comm[s + 1] = comm[s + 1] + stage[...]
            else:
                out_ref[...] = comm[s + 1] + stage[...]

    return pl.pallas_call(
        body,
        out_shape=jax.ShapeDtypeStruct((M, CHUNK), jnp.bfloat16),
        in_specs=[pl.BlockSpec(memory_space=pl.ANY)],
        out_specs=pl.BlockSpec(memory_space=pltpu.VMEM),
        scratch_shapes=[
            pltpu.VMEM((N_RING, M, CHUNK), jnp.bfloat16),
            pltpu.VMEM((M, CHUNK), jnp.bfloat16),
            pltpu.SemaphoreType.DMA,
            pltpu.SemaphoreType.DMA((N_RING - 1,)),
            pltpu.SemaphoreType.DMA((N_RING - 1,)),
        ],
        compiler_params=pltpu.CompilerParams(
            collective_id=0,
            vmem_limit_bytes=56 * 1024 * 1024,
        ),
    )(x_bf)


# device time: 296436 ns/iter; 1.1090x vs baseline; 1.1090x over previous
import jax
import jax.numpy as jnp
from jax import lax
from jax.experimental import pallas as pl
from jax.experimental.pallas import tpu as pltpu

N_RING = 4
M = 4096
H = M // 2
N_COLS = 4096
CHUNK = N_COLS // N_RING


def kernel(x):
    x2d = x[0]

    def body(
        x_hbm,
        out_ref,
        comm_cw,
        comm_ccw,
        stage_cw,
        stage_ccw,
        copy_sems,
        send_cw,
        recv_cw,
        send_ccw,
        recv_ccw,
    ):
        my_x = lax.axis_index("x")
        my_y = lax.axis_index("y")
        my_z = lax.axis_index("z")
        right = lax.rem(my_y + 1, N_RING)
        left = lax.rem(my_y + N_RING - 1, N_RING)

        barrier = pltpu.get_barrier_semaphore()
        for nbr in (left, right):
            pl.semaphore_signal(
                barrier, inc=1,
                device_id=(my_x, nbr, my_z),
                device_id_type=pl.DeviceIdType.MESH,
            )
        pl.semaphore_wait(barrier, 2)

        def stage_chunk(chunk_idx, top, dst, sem):
            row0 = 0 if top else H
            return pltpu.make_async_copy(
                x_hbm.at[pl.ds(row0, H), pl.ds(chunk_idx * CHUNK, CHUNK)],
                dst,
                sem,
            )

        c_cw0 = lax.rem(my_y + N_RING - 1, N_RING)
        c_ccw0 = lax.rem(my_y + 1, N_RING)
        cp0 = stage_chunk(c_cw0, True, stage_cw, copy_sems.at[0])
        cp1 = stage_chunk(c_ccw0, False, stage_ccw, copy_sems.at[1])
        cp0.start()
        cp1.start()
        cp0.wait()
        comm_cw[0] = stage_cw[...].astype(jnp.bfloat16)
        cp1.wait()
        comm_ccw[0] = stage_ccw[...].astype(jnp.bfloat16)

        for s in range(N_RING - 1):
            rdma_cw = pltpu.make_async_remote_copy(
                src_ref=comm_cw.at[s],
                dst_ref=comm_cw.at[s + 1],
                send_sem=send_cw.at[s],
                recv_sem=recv_cw.at[s],
                device_id=(my_x, right, my_z),
                device_id_type=pl.DeviceIdType.MESH,
            )
            rdma_ccw = pltpu.make_async_remote_copy(
                src_ref=comm_ccw.at[s],
                dst_ref=comm_ccw.at[s + 1],
                send_sem=send_ccw.at[s],
                recv_sem=recv_ccw.at[s],
                device_id=(my_x, left, my_z),
                device_id_type=pl.DeviceIdType.MESH,
            )
            rdma_cw.start()
            rdma_ccw.start()

            c_cw = lax.rem(my_y + 2 * N_RING - 2 - s, N_RING)
            c_ccw = lax.rem(my_y + 2 + s, N_RING)
            cpl0 = stage_chunk(c_cw, True, stage_cw, copy_sems.at[0])
            cpl1 = stage_chunk(c_ccw, False, stage_ccw, copy_sems.at[1])
            cpl0.start()
            cpl1.start()
            cpl0.wait()
            cpl1.wait()

            rdma_cw.wait()
            if s < N_RING - 2:
                comm_cw[s + 1] = comm_cw[s + 1] + stage_cw[...].astype(
                    jnp.bfloat16
                )
            else:
                out_ref[pl.ds(0, H), :] = comm_cw[s + 1] + stage_cw[
                    ...
                ].astype(jnp.bfloat16)
            rdma_ccw.wait()
            if s < N_RING - 2:
                comm_ccw[s + 1] = comm_ccw[s + 1] + stage_ccw[...].astype(
                    jnp.bfloat16
                )
            else:
                out_ref[pl.ds(H, H), :] = comm_ccw[s + 1] + stage_ccw[
                    ...
                ].astype(jnp.bfloat16)

    return pl.pallas_call(
        body,
        out_shape=jax.ShapeDtypeStruct((M, CHUNK), jnp.bfloat16),
        in_specs=[pl.BlockSpec(memory_space=pl.ANY)],
        out_specs=pl.BlockSpec(memory_space=pltpu.VMEM),
        scratch_shapes=[
            pltpu.VMEM((N_RING, H, CHUNK), jnp.bfloat16),
            pltpu.VMEM((N_RING, H, CHUNK), jnp.bfloat16),
            pltpu.VMEM((H, CHUNK), jnp.float32),
            pltpu.VMEM((H, CHUNK), jnp.float32),
            pltpu.SemaphoreType.DMA((2,)),
            pltpu.SemaphoreType.DMA((N_RING - 1,)),
            pltpu.SemaphoreType.DMA((N_RING - 1,)),
            pltpu.SemaphoreType.DMA((N_RING - 1,)),
            pltpu.SemaphoreType.DMA((N_RING - 1,)),
        ],
        compiler_params=pltpu.CompilerParams(
            collective_id=0,
            vmem_limit_bytes=60 * 1024 * 1024,
        ),
    )(x2d)


# device time: 163212 ns/iter; 2.0143x vs baseline; 1.8163x over previous
import jax
import jax.numpy as jnp
from jax import lax
from jax.experimental import pallas as pl
from jax.experimental.pallas import tpu as pltpu

N_Y = 4
N_Z = 4
M = 4096
R = M // N_Z
N_COLS = 4096
CHUNK = N_COLS // N_Y


def kernel(x):
    x2d = x[0]

    def body(
        x_hbm,
        out_ref,
        rs_comm,
        ag_comm,
        stage,
        copy_sem,
        rs_send,
        rs_recv,
        ag_send,
        ag_recv,
    ):
        my_x = lax.axis_index("x")
        my_y = lax.axis_index("y")
        my_z = lax.axis_index("z")
        y_right = lax.rem(my_y + 1, N_Y)
        y_left = lax.rem(my_y + N_Y - 1, N_Y)
        z_right = lax.rem(my_z + 1, N_Z)
        z_left = lax.rem(my_z + N_Z - 1, N_Z)

        barrier = pltpu.get_barrier_semaphore()
        for nbr_dev in (
            (my_x, y_left, my_z),
            (my_x, y_right, my_z),
            (my_x, my_y, z_left),
            (my_x, my_y, z_right),
        ):
            pl.semaphore_signal(
                barrier, inc=1,
                device_id=nbr_dev,
                device_id_type=pl.DeviceIdType.MESH,
            )
        pl.semaphore_wait(barrier, 4)

        row0 = my_z * R

        def stage_chunk(chunk_idx, dst):
            return pltpu.make_async_copy(
                x_hbm.at[pl.ds(row0, R), pl.ds(chunk_idx * CHUNK, CHUNK)],
                dst,
                copy_sem,
            )

        c0 = lax.rem(my_y + N_Y - 1, N_Y)
        cp = stage_chunk(c0, stage)
        cp.start()
        cp.wait()
        rs_comm[0] = stage[...].astype(jnp.bfloat16)

        for s in range(N_Y - 1):
            rdma = pltpu.make_async_remote_copy(
                src_ref=rs_comm.at[s],
                dst_ref=rs_comm.at[s + 1],
                send_sem=rs_send.at[s],
                recv_sem=rs_recv.at[s],
                device_id=(my_x, y_right, my_z),
                device_id_type=pl.DeviceIdType.MESH,
            )
            rdma.start()
            c = lax.rem(my_y + 2 * N_Y - 2 - s, N_Y)
            cpl = stage_chunk(c, stage)
            cpl.start()
            cpl.wait()
            rdma.wait()
            if s < N_Y - 2:
                rs_comm[s + 1] = rs_comm[s + 1] + stage[...].astype(
                    jnp.bfloat16
                )
            else:
                ag_comm[0] = rs_comm[s + 1] + stage[...].astype(jnp.bfloat16)

        out_ref[pl.ds(row0, R), :] = ag_comm[0]
        for t in range(N_Z - 1):
            rdma = pltpu.make_async_remote_copy(
                src_ref=ag_comm.at[t],
                dst_ref=ag_comm.at[t + 1],
                send_sem=ag_send.at[t],
                recv_sem=ag_recv.at[t],
                device_id=(my_x, my_y, z_right),
                device_id_type=pl.DeviceIdType.MESH,
            )
            rdma.start()
            rdma.wait()
            origin = lax.rem(my_z + 2 * N_Z - 1 - t, N_Z)
            out_ref[pl.ds(origin * R, R), :] = ag_comm[t + 1]

    return pl.pallas_call(
        body,
        out_shape=jax.ShapeDtypeStruct((M, CHUNK), jnp.bfloat16),
        in_specs=[pl.BlockSpec(memory_space=pl.ANY)],
        out_specs=pl.BlockSpec(memory_space=pltpu.VMEM),
        scratch_shapes=[
            pltpu.VMEM((N_Y, R, CHUNK), jnp.bfloat16),
            pltpu.VMEM((N_Z, R, CHUNK), jnp.bfloat16),
            pltpu.VMEM((R, CHUNK), jnp.float32),
            pltpu.SemaphoreType.DMA,
            pltpu.SemaphoreType.DMA((N_Y - 1,)),
            pltpu.SemaphoreType.DMA((N_Y - 1,)),
            pltpu.SemaphoreType.DMA((N_Z - 1,)),
            pltpu.SemaphoreType.DMA((N_Z - 1,)),
        ],
        compiler_params=pltpu.CompilerParams(
            collective_id=0,
            vmem_limit_bytes=48 * 1024 * 1024,
        ),
    )(x2d)


# device time: 109442 ns/iter; 3.0039x vs baseline; 1.4913x over previous
import jax
import jax.numpy as jnp
from jax import lax
from jax.experimental import pallas as pl
from jax.experimental.pallas import tpu as pltpu

N_Y = 4
N_Z = 4
N_X = 2
M = 4096
R = M // (N_Z * N_X)
N_COLS = 4096
CHUNK = N_COLS // N_Y


def kernel(x):
    x2d = x[0]

    def body(
        x_hbm,
        out_ref,
        rs_comm,
        ag_comm,
        stage,
        copy_sem,
        rs_send,
        rs_recv,
        ag_send,
        ag_recv,
        x_send,
        x_recv,
    ):
        my_x = lax.axis_index("x")
        my_y = lax.axis_index("y")
        my_z = lax.axis_index("z")
        other_x = lax.rem(my_x + 1, N_X)
        y_right = lax.rem(my_y + 1, N_Y)
        y_left = lax.rem(my_y + N_Y - 1, N_Y)
        z_right = lax.rem(my_z + 1, N_Z)
        z_left = lax.rem(my_z + N_Z - 1, N_Z)

        barrier = pltpu.get_barrier_semaphore()
        for nbr_dev in (
            (my_x, y_left, my_z),
            (my_x, y_right, my_z),
            (my_x, my_y, z_left),
            (my_x, my_y, z_right),
            (other_x, my_y, my_z),
        ):
            pl.semaphore_signal(
                barrier, inc=1,
                device_id=nbr_dev,
                device_id_type=pl.DeviceIdType.MESH,
            )
        pl.semaphore_wait(barrier, 5)

        my_b = N_X * my_z + my_x
        row0 = my_b * R

        def stage_chunk(chunk_idx, dst):
            return pltpu.make_async_copy(
                x_hbm.at[pl.ds(row0, R), pl.ds(chunk_idx * CHUNK, CHUNK)],
                dst,
                copy_sem,
            )

        c0 = lax.rem(my_y + N_Y - 1, N_Y)
        cp = stage_chunk(c0, stage)
        cp.start()
        cp.wait()
        rs_comm[0] = stage[...].astype(jnp.bfloat16)

        for s in range(N_Y - 1):
            rdma = pltpu.make_async_remote_copy(
                src_ref=rs_comm.at[s],
                dst_ref=rs_comm.at[s + 1],
                send_sem=rs_send.at[s],
                recv_sem=rs_recv.at[s],
                device_id=(my_x, y_right, my_z),
                device_id_type=pl.DeviceIdType.MESH,
            )
            rdma.start()
            c = lax.rem(my_y + 2 * N_Y - 2 - s, N_Y)
            cpl = stage_chunk(c, stage)
            cpl.start()
            cpl.wait()
            rdma.wait()
            if s < N_Y - 2:
                rs_comm[s + 1] = rs_comm[s + 1] + stage[...].astype(
                    jnp.bfloat16
                )
            else:
                ag_comm[0] = rs_comm[s + 1] + stage[...].astype(jnp.bfloat16)

        def x_rows(t):
            zp = lax.rem(my_z + N_Z - t, N_Z)
            return (N_X * zp + my_x) * R

        def x_rows_partner(t):
            zp = lax.rem(my_z + N_Z - t, N_Z)
            return (N_X * zp + other_x) * R

        x_rdmas = []
        for t in range(N_Z):
            send_desc = pltpu.make_async_remote_copy(
                src_ref=out_ref.at[pl.ds(x_rows(t), R), :],
                dst_ref=out_ref.at[pl.ds(x_rows(t), R), :],
                send_sem=x_send.at[t],
                recv_sem=x_recv.at[t],
                device_id=(other_x, my_y, my_z),
                device_id_type=pl.DeviceIdType.MESH,
            )
            recv_desc = pltpu.make_async_remote_copy(
                src_ref=out_ref.at[pl.ds(x_rows_partner(t), R), :],
                dst_ref=out_ref.at[pl.ds(x_rows_partner(t), R), :],
                send_sem=x_send.at[t],
                recv_sem=x_recv.at[t],
                device_id=(other_x, my_y, my_z),
                device_id_type=pl.DeviceIdType.MESH,
            )
            x_rdmas.append((send_desc, recv_desc))

        out_ref[pl.ds(row0, R), :] = ag_comm[0]
        x_rdmas[0][0].start()

        for t in range(N_Z - 1):
            rdma = pltpu.make_async_remote_copy(
                src_ref=ag_comm.at[t],
                dst_ref=ag_comm.at[t + 1],
                send_sem=ag_send.at[t],
                recv_sem=ag_recv.at[t],
                device_id=(my_x, my_y, z_right),
                device_id_type=pl.DeviceIdType.MESH,
            )
            rdma.start()
            rdma.wait()
            out_ref[pl.ds(x_rows(t + 1), R), :] = ag_comm[t + 1]
            x_rdmas[t + 1][0].start()

        for t in range(N_Z):
            x_rdmas[t][0].wait_send()
            x_rdmas[t][1].wait_recv()

    return pl.pallas_call(
        body,
        out_shape=jax.ShapeDtypeStruct((M, CHUNK), jnp.bfloat16),
        in_specs=[pl.BlockSpec(memory_space=pl.ANY)],
        out_specs=pl.BlockSpec(memory_space=pltpu.VMEM),
        scratch_shapes=[
            pltpu.VMEM((N_Y, R, CHUNK), jnp.bfloat16),
            pltpu.VMEM((N_Z, R, CHUNK), jnp.bfloat16),
            pltpu.VMEM((R, CHUNK), jnp.float32),
            pltpu.SemaphoreType.DMA,
            pltpu.SemaphoreType.DMA((N_Y - 1,)),
            pltpu.SemaphoreType.DMA((N_Y - 1,)),
            pltpu.SemaphoreType.DMA((N_Z - 1,)),
            pltpu.SemaphoreType.DMA((N_Z - 1,)),
            pltpu.SemaphoreType.DMA((N_Z,)),
            pltpu.SemaphoreType.DMA((N_Z,)),
        ],
        compiler_params=pltpu.CompilerParams(
            collective_id=0,
            vmem_limit_bytes=48 * 1024 * 1024,
        ),
    )(x2d)


# device time: 95593 ns/iter; 3.4391x vs baseline; 1.1449x over previous
import jax
import jax.numpy as jnp
from jax import lax
from jax.experimental import pallas as pl
from jax.experimental.pallas import tpu as pltpu

N_Y = 4
N_Z = 4
N_X = 2
M = 4096
R = M // (N_Z * N_X)
N_COLS = 4096
CHUNK = N_COLS // N_Y
P = 2
W = CHUNK // P


def kernel(x):
    x2d = x[0]

    def body(
        x_hbm,
        out_ref,
        rs_comm,
        ag_comm,
        stage,
        copy_sems,
        rs_send,
        rs_recv,
        ag_send,
        ag_recv,
        x_send,
        x_recv,
    ):
        my_x = lax.axis_index("x")
        my_y = lax.axis_index("y")
        my_z = lax.axis_index("z")
        other_x = lax.rem(my_x + 1, N_X)
        y_right = lax.rem(my_y + 1, N_Y)
        y_left = lax.rem(my_y + N_Y - 1, N_Y)
        z_right = lax.rem(my_z + 1, N_Z)
        z_left = lax.rem(my_z + N_Z - 1, N_Z)

        barrier = pltpu.get_barrier_semaphore()
        for nbr_dev in (
            (my_x, y_left, my_z),
            (my_x, y_right, my_z),
            (my_x, my_y, z_left),
            (my_x, my_y, z_right),
            (other_x, my_y, my_z),
        ):
            pl.semaphore_signal(
                barrier, inc=1,
                device_id=nbr_dev,
                device_id_type=pl.DeviceIdType.MESH,
            )
        pl.semaphore_wait(barrier, 5)

        my_b = N_X * my_z + my_x
        row0 = my_b * R

        def stage_dma(chunk_idx, h):
            return pltpu.make_async_copy(
                x_hbm.at[
                    pl.ds(row0, R),
                    pl.ds(chunk_idx * CHUNK + h * W, W),
                ],
                stage.at[h],
                copy_sems.at[h],
            )

        def rs_rdma(h, s):
            return pltpu.make_async_remote_copy(
                src_ref=rs_comm.at[h, s],
                dst_ref=rs_comm.at[h, s + 1],
                send_sem=rs_send.at[h, s],
                recv_sem=rs_recv.at[h, s],
                device_id=(my_x, y_right, my_z),
                device_id_type=pl.DeviceIdType.MESH,
            )

        def ag_rdma(h, t):
            return pltpu.make_async_remote_copy(
                src_ref=ag_comm.at[h, t],
                dst_ref=ag_comm.at[h, t + 1],
                send_sem=ag_send.at[h, t],
                recv_sem=ag_recv.at[h, t],
                device_id=(my_x, my_y, z_right),
                device_id_type=pl.DeviceIdType.MESH,
            )

        def blk_rows(t, xi):
            zp = lax.rem(my_z + N_Z - t, N_Z)
            return (N_X * zp + xi) * R

        def x_pair(h, t):
            send = pltpu.make_async_remote_copy(
                src_ref=out_ref.at[pl.ds(blk_rows(t, my_x), R),
                                   pl.ds(h * W, W)],
                dst_ref=out_ref.at[pl.ds(blk_rows(t, my_x), R),
                                   pl.ds(h * W, W)],
                send_sem=x_send.at[h, t],
                recv_sem=x_recv.at[h, t],
                device_id=(other_x, my_y, my_z),
                device_id_type=pl.DeviceIdType.MESH,
            )
            recv = pltpu.make_async_remote_copy(
                src_ref=out_ref.at[pl.ds(blk_rows(t, other_x), R),
                                   pl.ds(h * W, W)],
                dst_ref=out_ref.at[pl.ds(blk_rows(t, other_x), R),
                                   pl.ds(h * W, W)],
                send_sem=x_send.at[h, t],
                recv_sem=x_recv.at[h, t],
                device_id=(other_x, my_y, my_z),
                device_id_type=pl.DeviceIdType.MESH,
            )
            return send, recv

        x_pairs = [[x_pair(h, t) for t in range(N_Z)] for h in range(P)]

        def store_and_xsend(h, t):
            if t == 0:
                out_ref[pl.ds(row0, R), pl.ds(h * W, W)] = ag_comm[h, 0]
            else:
                out_ref[
                    pl.ds(blk_rows(t, my_x), R), pl.ds(h * W, W)
                ] = ag_comm[h, t]
            x_pairs[h][t][0].start()

        c_seed = lax.rem(my_y + N_Y - 1, N_Y)

        def c_hop(s):
            return lax.rem(my_y + 2 * N_Y - 2 - s, N_Y)

        def rs_accum(h, s):
            contrib = stage[h].astype(jnp.bfloat16)
            if s < N_Y - 2:
                rs_comm[h, s + 1] = rs_comm[h, s + 1] + contrib
            else:
                ag_comm[h, 0] = rs_comm[h, s + 1] + contrib

        cp = stage_dma(c_seed, 0)
        cp.start()
        cp1 = stage_dma(c_seed, 1)
        cp1.start()
        cp.wait()
        rs_comm[0, 0] = stage[0].astype(jnp.bfloat16)
        cp1.wait()
        rs_comm[1, 0] = stage[1].astype(jnp.bfloat16)

        rs0 = [rs_rdma(0, s) for s in range(N_Y - 1)]
        for s in range(N_Y - 1):
            rs0[s].start()
            cpl = stage_dma(c_hop(s), 0)
            cpl.start()
            cpl.wait()
            rs0[s].wait()
            rs_accum(0, s)

        store_and_xsend(0, 0)
        ag0 = [ag_rdma(0, t) for t in range(N_Z - 1)]
        rs1 = [rs_rdma(1, s) for s in range(N_Y - 1)]
        ag0[0].start()
        for s in range(N_Y - 1):
            rs1[s].start()
            cpl = stage_dma(c_hop(s), 1)
            cpl.start()
            cpl.wait()
            rs1[s].wait()
            rs_accum(1, s)
            ag0[s].wait()
            store_and_xsend(0, s + 1)
            if s < N_Z - 2:
                ag0[s + 1].start()

        store_and_xsend(1, 0)
        ag1 = [ag_rdma(1, t) for t in range(N_Z - 1)]
        ag1[0].start()
        for t in range(N_Z - 1):
            ag1[t].wait()
            store_and_xsend(1, t + 1)
            if t < N_Z - 2:
                ag1[t + 1].start()

        for h in range(P):
            for t in range(N_Z):
                x_pairs[h][t][0].wait_send()
                x_pairs[h][t][1].wait_recv()

    return pl.pallas_call(
        body,
        out_shape=jax.ShapeDtypeStruct((M, CHUNK), jnp.bfloat16),
        in_specs=[pl.BlockSpec(memory_space=pl.ANY)],
        out_specs=pl.BlockSpec(memory_space=pltpu.VMEM),
        scratch_shapes=[
            pltpu.VMEM((P, N_Y, R, W), jnp.bfloat16),
            pltpu.VMEM((P, N_Z, R, W), jnp.bfloat16),
            pltpu.VMEM((P, R, W), jnp.float32),
            pltpu.SemaphoreType.DMA((P,)),
            pltpu.SemaphoreType.DMA((P, N_Y - 1)),
            pltpu.SemaphoreType.DMA((P, N_Y - 1)),
            pltpu.SemaphoreType.DMA((P, N_Z - 1)),
            pltpu.SemaphoreType.DMA((P, N_Z - 1)),
            pltpu.SemaphoreType.DMA((P, N_Z)),
            pltpu.SemaphoreType.DMA((P, N_Z)),
        ],
        compiler_params=pltpu.CompilerParams(
            collective_id=0,
            vmem_limit_bytes=48 * 1024 * 1024,
        ),
    )(x2d)


# device time: 94064 ns/iter; 3.4950x vs baseline; 1.0163x over previous
import jax
import jax.numpy as jnp
from jax import lax
from jax.experimental import pallas as pl
from jax.experimental.pallas import tpu as pltpu

N_Y = 4
N_Z = 4
N_X = 2
M = 4096
R = M // (N_Z * N_X)
N_COLS = 4096
CHUNK = N_COLS // N_Y
P = 4
W = CHUNK // P


def kernel(x):
    x2d = x[0]

    def body(
        x_hbm,
        out_ref,
        rs_comm,
        ag_comm,
        stage,
        copy_sems,
        rs_send,
        rs_recv,
        ag_send,
        ag_recv,
        x_send,
        x_recv,
    ):
        my_x = lax.axis_index("x")
        my_y = lax.axis_index("y")
        my_z = lax.axis_index("z")
        other_x = lax.rem(my_x + 1, N_X)
        y_right = lax.rem(my_y + 1, N_Y)
        y_left = lax.rem(my_y + N_Y - 1, N_Y)
        z_right = lax.rem(my_z + 1, N_Z)
        z_left = lax.rem(my_z + N_Z - 1, N_Z)

        barrier = pltpu.get_barrier_semaphore()
        for nbr_dev in (
            (my_x, y_left, my_z),
            (my_x, y_right, my_z),
            (my_x, my_y, z_left),
            (my_x, my_y, z_right),
            (other_x, my_y, my_z),
        ):
            pl.semaphore_signal(
                barrier, inc=1,
                device_id=nbr_dev,
                device_id_type=pl.DeviceIdType.MESH,
            )
        pl.semaphore_wait(barrier, 5)

        my_b = N_X * my_z + my_x
        row0 = my_b * R

        def stage_dma(chunk_idx, h):
            return pltpu.make_async_copy(
                x_hbm.at[
                    pl.ds(row0, R),
                    pl.ds(chunk_idx * CHUNK + h * W, W),
                ],
                stage.at[h],
                copy_sems.at[h],
            )

        def rs_rdma(h, s):
            return pltpu.make_async_remote_copy(
                src_ref=rs_comm.at[h, s],
                dst_ref=rs_comm.at[h, s + 1],
                send_sem=rs_send.at[h, s],
                recv_sem=rs_recv.at[h, s],
                device_id=(my_x, y_right, my_z),
                device_id_type=pl.DeviceIdType.MESH,
            )

        def ag_rdma(h, t):
            return pltpu.make_async_remote_copy(
                src_ref=ag_comm.at[h, t],
                dst_ref=ag_comm.at[h, t + 1],
                send_sem=ag_send.at[h, t],
                recv_sem=ag_recv.at[h, t],
                device_id=(my_x, my_y, z_right),
                device_id_type=pl.DeviceIdType.MESH,
            )

        def blk_rows(t, xi):
            zp = lax.rem(my_z + N_Z - t, N_Z)
            return (N_X * zp + xi) * R

        def x_pair(h, t):
            send = pltpu.make_async_remote_copy(
                src_ref=out_ref.at[pl.ds(blk_rows(t, my_x), R),
                                   pl.ds(h * W, W)],
                dst_ref=out_ref.at[pl.ds(blk_rows(t, my_x), R),
                                   pl.ds(h * W, W)],
                send_sem=x_send.at[h, t],
                recv_sem=x_recv.at[h, t],
                device_id=(other_x, my_y, my_z),
                device_id_type=pl.DeviceIdType.MESH,
            )
            recv = pltpu.make_async_remote_copy(
                src_ref=out_ref.at[pl.ds(blk_rows(t, other_x), R),
                                   pl.ds(h * W, W)],
                dst_ref=out_ref.at[pl.ds(blk_rows(t, other_x), R),
                                   pl.ds(h * W, W)],
                send_sem=x_send.at[h, t],
                recv_sem=x_recv.at[h, t],
                device_id=(other_x, my_y, my_z),
                device_id_type=pl.DeviceIdType.MESH,
            )
            return send, recv

        x_pairs = [[x_pair(h, t) for t in range(N_Z)] for h in range(P)]

        def store_and_xsend(h, t):
            if t == 0:
                out_ref[pl.ds(row0, R), pl.ds(h * W, W)] = ag_comm[h, 0]
            else:
                out_ref[
                    pl.ds(blk_rows(t, my_x), R), pl.ds(h * W, W)
                ] = ag_comm[h, t]
            x_pairs[h][t][0].start()

        c_seed = lax.rem(my_y + N_Y - 1, N_Y)

        def c_hop(s):
            return lax.rem(my_y + 2 * N_Y - 2 - s, N_Y)

        def rs_accum(h, s):
            contrib = stage[h].astype(jnp.bfloat16)
            if s < N_Y - 2:
                rs_comm[h, s + 1] = rs_comm[h, s + 1] + contrib
            else:
                ag_comm[h, 0] = rs_comm[h, s + 1] + contrib

        seed_dmas = []
        for h in range(P):
            d = stage_dma(c_seed, h)
            d.start()
            seed_dmas.append(d)
        for h in range(P):
            seed_dmas[h].wait()
            rs_comm[h, 0] = stage[h].astype(jnp.bfloat16)

        rs_d = [[rs_rdma(h, s) for s in range(N_Y - 1)] for h in range(P)]
        ag_d = [[ag_rdma(h, t) for t in range(N_Z - 1)] for h in range(P)]

        dmas = []
        for h in range(P):
            rs_d[h][0].start()
            d = stage_dma(c_hop(0), h)
            d.start()
            dmas.append(d)

        for s in range(N_Y - 1):
            for h in range(P):
                dmas[h].wait()
                rs_d[h][s].wait()
                rs_accum(h, s)
                if s < N_Y - 2:
                    rs_d[h][s + 1].start()
                    d = stage_dma(c_hop(s + 1), h)
                    d.start()
                    dmas[h] = d
                else:
                    store_and_xsend(h, 0)
                    ag_d[h][0].start()

        for t in range(N_Z - 1):
            for h in range(P):
                ag_d[h][t].wait()
                store_and_xsend(h, t + 1)
                if t < N_Z - 2:
                    ag_d[h][t + 1].start()

        for h in range(P):
            for t in range(N_Z):
                x_pairs[h][t][0].wait_send()
                x_pairs[h][t][1].wait_recv()

    return pl.pallas_call(
        body,
        out_shape=jax.ShapeDtypeStruct((M, CHUNK), jnp.bfloat16),
        in_specs=[pl.BlockSpec(memory_space=pl.ANY)],
        out_specs=pl.BlockSpec(memory_space=pltpu.VMEM),
        scratch_shapes=[
            pltpu.VMEM((P, N_Y, R, W), jnp.bfloat16),
            pltpu.VMEM((P, N_Z, R, W), jnp.bfloat16),
            pltpu.VMEM((P, R, W), jnp.float32),
            pltpu.SemaphoreType.DMA((P,)),
            pltpu.SemaphoreType.DMA((P, N_Y - 1)),
            pltpu.SemaphoreType.DMA((P, N_Y - 1)),
            pltpu.SemaphoreType.DMA((P, N_Z - 1)),
            pltpu.SemaphoreType.DMA((P, N_Z - 1)),
            pltpu.SemaphoreType.DMA((P, N_Z)),
            pltpu.SemaphoreType.DMA((P, N_Z)),
        ],
        compiler_params=pltpu.CompilerParams(
            collective_id=0,
            vmem_limit_bytes=48 * 1024 * 1024,
        ),
    )(x2d)


# device time: 88690 ns/iter; 3.7068x vs baseline; 1.0606x over previous
import jax
import jax.numpy as jnp
from jax import lax
from jax.experimental import pallas as pl
from jax.experimental.pallas import tpu as pltpu

N_Y = 4
N_Z = 4
N_X = 2
M = 4096
R = M // (N_Z * N_X)
N_COLS = 4096
CHUNK = N_COLS // N_Y
P = 4
W = CHUNK // P
OFF = 2
N_EV = 7


def kernel(x):
    x2d = x[0]

    def body(
        x_hbm,
        out_ref,
        rs_comm,
        stage,
        copy_sems,
        rs_send,
        rs_recv,
        ag_send,
        ag_recv,
        x_send,
        x_recv,
    ):
        my_x = lax.axis_index("x")
        my_y = lax.axis_index("y")
        my_z = lax.axis_index("z")
        other_x = lax.rem(my_x + 1, N_X)
        y_right = lax.rem(my_y + 1, N_Y)
        y_left = lax.rem(my_y + N_Y - 1, N_Y)
        z_right = lax.rem(my_z + 1, N_Z)
        z_left = lax.rem(my_z + N_Z - 1, N_Z)

        barrier = pltpu.get_barrier_semaphore()
        for nbr_dev in (
            (my_x, y_left, my_z),
            (my_x, y_right, my_z),
            (my_x, my_y, z_left),
            (my_x, my_y, z_right),
            (other_x, my_y, my_z),
        ):
            pl.semaphore_signal(
                barrier, inc=1,
                device_id=nbr_dev,
                device_id_type=pl.DeviceIdType.MESH,
            )
        pl.semaphore_wait(barrier, 5)

        my_b = N_X * my_z + my_x
        row0 = my_b * R

        def stage_dma(chunk_idx, h):
            return pltpu.make_async_copy(
                x_hbm.at[
                    pl.ds(row0, R),
                    pl.ds(chunk_idx * CHUNK + h * W, W),
                ],
                stage.at[h],
                copy_sems.at[h],
            )

        def rs_rdma(h, s):
            return pltpu.make_async_remote_copy(
                src_ref=rs_comm.at[h, s],
                dst_ref=rs_comm.at[h, s + 1],
                send_sem=rs_send.at[h, s],
                recv_sem=rs_recv.at[h, s],
                device_id=(my_x, y_right, my_z),
                device_id_type=pl.DeviceIdType.MESH,
            )

        def blk_rows(t, xi):
            zp = lax.rem(my_z + N_Z - t, N_Z)
            return (N_X * zp + xi) * R

        def ag_rdma(h, t):
            sl = (pl.ds(blk_rows(t, my_x), R), pl.ds(h * W, W))
            return pltpu.make_async_remote_copy(
                src_ref=out_ref.at[sl],
                dst_ref=out_ref.at[sl],
                send_sem=ag_send.at[h, t],
                recv_sem=ag_recv.at[h, t],
                device_id=(my_x, my_y, z_right),
                device_id_type=pl.DeviceIdType.MESH,
            )

        def x_pair(h, t):
            sl_mine = (pl.ds(blk_rows(t, my_x), R), pl.ds(h * W, W))
            sl_theirs = (pl.ds(blk_rows(t, other_x), R), pl.ds(h * W, W))
            send = pltpu.make_async_remote_copy(
                src_ref=out_ref.at[sl_mine],
                dst_ref=out_ref.at[sl_mine],
                send_sem=x_send.at[h, t],
                recv_sem=x_recv.at[h, t],
                device_id=(other_x, my_y, my_z),
                device_id_type=pl.DeviceIdType.MESH,
            )
            recv = pltpu.make_async_remote_copy(
                src_ref=out_ref.at[sl_theirs],
                dst_ref=out_ref.at[sl_theirs],
                send_sem=x_send.at[h, t],
                recv_sem=x_recv.at[h, t],
                device_id=(other_x, my_y, my_z),
                device_id_type=pl.DeviceIdType.MESH,
            )
            return send, recv

        x_pairs = [[x_pair(h, t) for t in range(N_Z)] for h in range(P)]

        c_seed = lax.rem(my_y + N_Y - 1, N_Y)

        def c_hop(s):
            return lax.rem(my_y + 2 * N_Y - 2 - s, N_Y)

        seed_dmas = []
        for h in range(P):
            d = stage_dma(c_seed, h)
            d.start()
            seed_dmas.append(d)
        for h in range(P):
            seed_dmas[h].wait()
            rs_comm[h, 0] = stage[h].astype(jnp.bfloat16)

        rs_d = [[rs_rdma(h, s) for s in range(N_Y - 1)] for h in range(P)]
        ag_d = [[ag_rdma(h, t) for t in range(N_Z - 1)] for h in range(P)]
        dmas = [None] * P

        def event(h, e):
            if e == 0:
                rs_d[h][0].start()
                d = stage_dma(c_hop(0), h)
                d.start()
                dmas[h] = d
            elif e <= 3:
                s = e - 1
                dmas[h].wait()
                rs_d[h][s].wait()
                contrib = stage[h].astype(jnp.bfloat16)
                if s < N_Y - 2:
                    rs_comm[h, s + 1] = rs_comm[h, s + 1] + contrib
                    rs_d[h][s + 1].start()
                    d = stage_dma(c_hop(s + 1), h)
                    d.start()
                    dmas[h] = d
                else:
                    out_ref[pl.ds(row0, R), pl.ds(h * W, W)] = (
                        rs_comm[h, s + 1] + contrib
                    )
                    x_pairs[h][0][0].start()
                    ag_d[h][0].start()
            else:
                t = e - 4
                ag_d[h][t].wait()
                x_pairs[h][t + 1][0].start()
                if t < N_Z - 2:
                    ag_d[h][t + 1].start()

        for k in range(N_EV + (P - 1) * OFF):
            for h in range(P):
                e = k - h * OFF
                if 0 <= e < N_EV:
                    event(h, e)

        for h in range(P):
            for t in range(N_Z):
                x_pairs[h][t][0].wait_send()
                x_pairs[h][t][1].wait_recv()

    return pl.pallas_call(
        body,
        out_shape=jax.ShapeDtypeStruct((M, CHUNK), jnp.bfloat16),
        in_specs=[pl.BlockSpec(memory_space=pl.ANY)],
        out_specs=pl.BlockSpec(memory_space=pltpu.VMEM),
        scratch_shapes=[
            pltpu.VMEM((P, N_Y, R, W), jnp.bfloat16),
            pltpu.VMEM((P, R, W), jnp.float32),
            pltpu.SemaphoreType.DMA((P,)),
            pltpu.SemaphoreType.DMA((P, N_Y - 1)),
            pltpu.SemaphoreType.DMA((P, N_Y - 1)),
            pltpu.SemaphoreType.DMA((P, N_Z - 1)),
            pltpu.SemaphoreType.DMA((P, N_Z - 1)),
            pltpu.SemaphoreType.DMA((P, N_Z)),
            pltpu.SemaphoreType.DMA((P, N_Z)),
        ],
        compiler_params=pltpu.CompilerParams(
            collective_id=0,
            vmem_limit_bytes=48 * 1024 * 1024,
        ),
    )(x2d)


# device time: 86692 ns/iter; 3.7922x vs baseline; 1.0230x over previous
import jax
import jax.numpy as jnp
from jax import lax
from jax.experimental import pallas as pl
from jax.experimental.pallas import tpu as pltpu

N_Y = 4
N_Z = 4
N_X = 2
M = 4096
R = M // (N_Z * N_X)
N_COLS = 4096
CHUNK = N_COLS // N_Y
P = 4
W = CHUNK // P
OFF = 1
N_EV = 7


def kernel(x):
    x2d = x[0]

    def body(
        x_hbm,
        out_ref,
        rs_comm,
        stage,
        copy_sems,
        rs_send,
        rs_recv,
        ag_send,
        ag_recv,
        x_send,
        x_recv,
    ):
        my_x = lax.axis_index("x")
        my_y = lax.axis_index("y")
        my_z = lax.axis_index("z")
        other_x = lax.rem(my_x + 1, N_X)
        y_right = lax.rem(my_y + 1, N_Y)
        y_left = lax.rem(my_y + N_Y - 1, N_Y)
        z_right = lax.rem(my_z + 1, N_Z)
        z_left = lax.rem(my_z + N_Z - 1, N_Z)

        barrier = pltpu.get_barrier_semaphore()
        for nbr_dev in (
            (my_x, y_left, my_z),
            (my_x, y_right, my_z),
            (my_x, my_y, z_left),
            (my_x, my_y, z_right),
            (other_x, my_y, my_z),
        ):
            pl.semaphore_signal(
                barrier, inc=1,
                device_id=nbr_dev,
                device_id_type=pl.DeviceIdType.MESH,
            )
        pl.semaphore_wait(barrier, 5)

        my_b = N_X * my_z + my_x
        row0 = my_b * R

        def stage_dma(chunk_idx, h):
            return pltpu.make_async_copy(
                x_hbm.at[
                    pl.ds(row0, R),
                    pl.ds(chunk_idx * CHUNK + h * W, W),
                ],
                stage.at[h],
                copy_sems.at[h],
            )

        def rs_rdma(h, s):
            return pltpu.make_async_remote_copy(
                src_ref=rs_comm.at[h, s],
                dst_ref=rs_comm.at[h, s + 1],
                send_sem=rs_send.at[h, s],
                recv_sem=rs_recv.at[h, s],
                device_id=(my_x, y_right, my_z),
                device_id_type=pl.DeviceIdType.MESH,
            )

        def blk_rows(t, xi):
            zp = lax.rem(my_z + N_Z - t, N_Z)
            return (N_X * zp + xi) * R

        def ag_rdma(h, t):
            sl = (pl.ds(blk_rows(t, my_x), R), pl.ds(h * W, W))
            return pltpu.make_async_remote_copy(
                src_ref=out_ref.at[sl],
                dst_ref=out_ref.at[sl],
                send_sem=ag_send.at[h, t],
                recv_sem=ag_recv.at[h, t],
                device_id=(my_x, my_y, z_right),
                device_id_type=pl.DeviceIdType.MESH,
            )

        def x_pair(h, t):
            sl_mine = (pl.ds(blk_rows(t, my_x), R), pl.ds(h * W, W))
            sl_theirs = (pl.ds(blk_rows(t, other_x), R), pl.ds(h * W, W))
            send = pltpu.make_async_remote_copy(
                src_ref=out_ref.at[sl_mine],
                dst_ref=out_ref.at[sl_mine],
                send_sem=x_send.at[h, t],
                recv_sem=x_recv.at[h, t],
                device_id=(other_x, my_y, my_z),
                device_id_type=pl.DeviceIdType.MESH,
            )
            recv = pltpu.make_async_remote_copy(
                src_ref=out_ref.at[sl_theirs],
                dst_ref=out_ref.at[sl_theirs],
                send_sem=x_send.at[h, t],
                recv_sem=x_recv.at[h, t],
                device_id=(other_x, my_y, my_z),
                device_id_type=pl.DeviceIdType.MESH,
            )
            return send, recv

        x_pairs = [[x_pair(h, t) for t in range(N_Z)] for h in range(P)]

        c_seed = lax.rem(my_y + N_Y - 1, N_Y)

        def c_hop(s):
            return lax.rem(my_y + 2 * N_Y - 2 - s, N_Y)

        seed_dmas = []
        for h in range(P):
            d = stage_dma(c_seed, h)
            d.start()
            seed_dmas.append(d)
        for h in range(P):
            seed_dmas[h].wait()
            rs_comm[h, 0] = stage[h].astype(jnp.bfloat16)

        rs_d = [[rs_rdma(h, s) for s in range(N_Y - 1)] for h in range(P)]
        ag_d = [[ag_rdma(h, t) for t in range(N_Z - 1)] for h in range(P)]
        dmas = [None] * P

        def event(h, e):
            if e == 0:
                rs_d[h][0].start()
                d = stage_dma(c_hop(0), h)
                d.start()
                dmas[h] = d
            elif e <= 3:
                s = e - 1
                dmas[h].wait()
                rs_d[h][s].wait()
                contrib = stage[h].astype(jnp.bfloat16)
                if s < N_Y - 2:
                    rs_comm[h, s + 1] = rs_comm[h, s + 1] + contrib
                    rs_d[h][s + 1].start()
                    d = stage_dma(c_hop(s + 1), h)
                    d.start()
                    dmas[h] = d
                else:
                    out_ref[pl.ds(row0, R), pl.ds(h * W, W)] = (
                        rs_comm[h, s + 1] + contrib
                    )
                    x_pairs[h][0][0].start()
                    ag_d[h][0].start()
            else:
                t = e - 4
                ag_d[h][t].wait()
                x_pairs[h][t + 1][0].start()
                if t < N_Z - 2:
                    ag_d[h][t + 1].start()

        for k in range(N_EV + (P - 1) * OFF):
            for h in range(P):
                e = k - h * OFF
                if 0 <= e < N_EV:
                    event(h, e)

        for h in range(P):
            for t in range(N_Z):
                x_pairs[h][t][0].wait_send()
                x_pairs[h][t][1].wait_recv()

    return pl.pallas_call(
        body,
        out_shape=jax.ShapeDtypeStruct((M, CHUNK), jnp.bfloat16),
        in_specs=[pl.BlockSpec(memory_space=pl.ANY)],
        out_specs=pl.BlockSpec(memory_space=pltpu.VMEM),
        scratch_shapes=[
            pltpu.VMEM((P, N_Y, R, W), jnp.bfloat16),
            pltpu.VMEM((P, R, W), jnp.float32),
            pltpu.SemaphoreType.DMA((P,)),
            pltpu.SemaphoreType.DMA((P, N_Y - 1)),
            pltpu.SemaphoreType.DMA((P, N_Y - 1)),
            pltpu.SemaphoreType.DMA((P, N_Z - 1)),
            pltpu.SemaphoreType.DMA((P, N_Z - 1)),
            pltpu.SemaphoreType.DMA((P, N_Z)),
            pltpu.SemaphoreType.DMA((P, N_Z)),
        ],
        compiler_params=pltpu.CompilerParams(
            collective_id=0,
            vmem_limit_bytes=48 * 1024 * 1024,
        ),
    )(x2d)
